# baseline (speedup 1.0000x reference)
import functools

import jax
import jax.numpy as jnp
from jax import lax
from jax.experimental import pallas as pl
from jax.experimental.pallas import tpu as pltpu
from jax.experimental.pallas import tpu_sc as plsc

B = 4096
L = 200
D = 128
N = B * L
NC = 2
NS = 16
NW = NC * NS
PER_W = N // NW
CHUNK = 200
NCHUNK = PER_W // CHUNK
NPAIR = NCHUNK // 2

_mesh = plsc.VectorSubcoreMesh(core_axis_name="c", subcore_axis_name="s")


@functools.partial(
    pl.kernel,
    mesh=_mesh,
    out_type=jax.ShapeDtypeStruct((N, D), jnp.float32),
    scratch_types=(
        [pltpu.VMEM((PER_W,), jnp.int32)]
        + [pltpu.VMEM((CHUNK, D), jnp.float32) for _ in range(2)]
        + [pltpu.VMEM_SHARED((NS * CHUNK, D), jnp.float32) for _ in range(2)]
        + [pltpu.SemaphoreType.DMA for _ in range(6)]
    ),
)
def _gather_kernel(idx_hbm, table_hbm, out_hbm, idx_v, *rest):
    b = rest[0:2]
    sp = rest[2:4]
    gs = rest[4:6]
    cs = rest[6:8]
    ws = rest[8:10]
    sid = lax.axis_index("s")
    wid = sid * NC + lax.axis_index("c")
    base = wid * PER_W
    s = [sp[i].at[pl.ds(sid * CHUNK, CHUNK)] for i in range(2)]
    pltpu.sync_copy(idx_hbm.at[pl.ds(base, PER_W)], idx_v)
    for p in range(2):
        pltpu.async_copy(
            table_hbm.at[idx_v.at[pl.ds(p * CHUNK, CHUNK)]], b[p], gs[p])

    def body(j, carry):
        g0 = j * 2
        for p in range(2):
            g = g0 + p
            pltpu.make_async_copy(
                table_hbm.at[idx_v.at[pl.ds(g * CHUNK, CHUNK)]], b[p],
                gs[p]).wait()

            @pl.when(j > 0)
            def _(p=p, g=g):
                pltpu.make_async_copy(
                    s[p], out_hbm.at[pl.ds(base + (g - 2) * CHUNK, CHUNK)],
                    ws[p]).wait()

            pltpu.async_copy(b[p], s[p], cs[p])

            def tail(issue_gather, p=p, g=g):
                pltpu.make_async_copy(b[1 - p], s[1 - p], cs[1 - p]).wait()
                pltpu.async_copy(
                    s[1 - p],
                    out_hbm.at[pl.ds(base + (g - 1) * CHUNK, CHUNK)],
                    ws[1 - p])

                def issue(p=p, g=g):
                    pltpu.async_copy(
                        table_hbm.at[idx_v.at[pl.ds((g + 1) * CHUNK, CHUNK)]],
                        b[1 - p], gs[1 - p])

                if issue_gather is None:
                    issue()
                else:
                    pl.when(issue_gather)(issue)

            if p == 0:
                pl.when(j > 0)(functools.partial(tail, None))
            else:
                tail(j + 1 < NPAIR)
        return carry

    lax.fori_loop(0, NPAIR, body, 0)
    gl = NCHUNK - 1
    pltpu.make_async_copy(b[gl % 2], s[gl % 2], cs[gl % 2]).wait()
    pltpu.async_copy(
        s[gl % 2], out_hbm.at[pl.ds(base + gl * CHUNK, CHUNK)], ws[gl % 2])
    for g in (NCHUNK - 2, NCHUNK - 1):
        pltpu.make_async_copy(
            s[g % 2], out_hbm.at[pl.ds(base + g * CHUNK, CHUNK)],
            ws[g % 2]).wait()


def kernel(x, table):
    out = _gather_kernel(x.reshape(-1), table)
    return out.reshape(B, L, D)
